# baseline (speedup 1.0000x reference)
import functools

import jax
import jax.numpy as jnp
from jax import lax
from jax.experimental import pallas as pl
from jax.experimental.pallas import tpu as pltpu
from jax.experimental.pallas import tpu_sc as plsc

BATCH = 16384
D = 64
CR = 128


def kernel(user, item_i, item_j, embed_user, embed_item):
    info = plsc.get_sparse_core_info()
    NC, NS = info.num_cores, info.num_subcores
    NW = NC * NS
    BPW = BATCH // NW
    NCHK = BPW // CR

    u2 = user.reshape(NW, BPW)
    i2 = item_i.reshape(NW, BPW)
    j2 = item_j.reshape(NW, BPW)

    eu_b = embed_user.astype(jnp.bfloat16)
    ei_b = embed_item.astype(jnp.bfloat16)

    mesh = plsc.VectorSubcoreMesh(core_axis_name="c", subcore_axis_name="s")

    @functools.partial(
        pl.kernel,
        out_type=(jax.ShapeDtypeStruct((BATCH,), jnp.float32),
                  jax.ShapeDtypeStruct((BATCH,), jnp.float32)),
        mesh=mesh,
        compiler_params=pltpu.CompilerParams(needs_layout_passes=False),
        scratch_types=[
            pltpu.VMEM((BPW,), jnp.int32),
            pltpu.VMEM((BPW,), jnp.int32),
            pltpu.VMEM((BPW,), jnp.int32),
            pltpu.VMEM((CR, D), jnp.int32),
            pltpu.VMEM((CR, D), jnp.int32),
            pltpu.VMEM((CR, D), jnp.int32),
            pltpu.VMEM((BPW,), jnp.float32),
            pltpu.VMEM((BPW,), jnp.float32),
            pltpu.SemaphoreType.DMA,
        ],
    )
    def bprmf(u_hbm, ii_hbm, ij_hbm, eu_hbm, ei_hbm, oi_hbm, oj_hbm,
              ru_v, ri_v, rj_v, gu_v, gi_v, gj_v, oi_v, oj_v, sem):
        wid = lax.axis_index("s") * NC + lax.axis_index("c")
        pltpu.sync_copy(u_hbm.at[wid], ru_v)
        pltpu.sync_copy(ii_hbm.at[wid], ri_v)
        pltpu.sync_copy(ij_hbm.at[wid], rj_v)

        eu_w = eu_hbm.bitcast(jnp.int32)
        ei_w = ei_hbm.bitcast(jnp.int32)

        iota16 = lax.iota(jnp.int32, 16)

        def chunk_body(c, carry):
            def fire_body(g, carry2):
                base = pl.multiple_of(c * CR + g * 16, 16)
                uvec = ru_v[pl.ds(base, 16)] >> 1
                ivec = ri_v[pl.ds(base, 16)] >> 1
                jvec = rj_v[pl.ds(base, 16)] >> 1
                for l in range(16):
                    k = g * 16 + l
                    pltpu.async_copy(eu_w.at[uvec[l]], gu_v.at[k], sem)
                    pltpu.async_copy(ei_w.at[ivec[l]], gi_v.at[k], sem)
                    pltpu.async_copy(ei_w.at[jvec[l]], gj_v.at[k], sem)
                return carry2

            lax.fori_loop(0, CR // 16, fire_body, 0)
            pltpu.make_async_copy(eu_w.at[pl.ds(0, CR)], gu_v, sem).wait()
            pltpu.make_async_copy(eu_w.at[pl.ds(0, CR)], gi_v, sem).wait()
            pltpu.make_async_copy(eu_w.at[pl.ds(0, CR)], gj_v, sem).wait()

            himask = jnp.full((16,), -65536, jnp.int32)

            def group_body(g, carry2):
                off = pl.multiple_of(c * CR + g * 16, 16)
                uodd = (ru_v[pl.ds(off, 16)] & 1) == 1
                iodd = (ri_v[pl.ds(off, 16)] & 1) == 1
                jodd = (rj_v[pl.ds(off, 16)] & 1) == 1
                items = g * 16 + iota16
                acc_i = jnp.zeros((16,), jnp.float32)
                acc_j = jnp.zeros((16,), jnp.float32)
                for d in range(D):
                    cols = jnp.full((16,), d, jnp.int32)
                    pu = plsc.load_gather(gu_v, [items, cols])
                    pi = plsc.load_gather(gi_v, [items, cols])
                    pj = plsc.load_gather(gj_v, [items, cols])
                    uu = plsc.bitcast(
                        jnp.where(uodd, pu & himask, pu << 16), jnp.float32)
                    vi = plsc.bitcast(
                        jnp.where(iodd, pi & himask, pi << 16), jnp.float32)
                    vj = plsc.bitcast(
                        jnp.where(jodd, pj & himask, pj << 16), jnp.float32)
                    acc_i = acc_i + uu * vi
                    acc_j = acc_j + uu * vj
                off = pl.multiple_of(c * CR + g * 16, 16)
                oi_v[pl.ds(off, 16)] = acc_i
                oj_v[pl.ds(off, 16)] = acc_j
                return carry2

            lax.fori_loop(0, CR // 16, group_body, 0)
            return carry

        lax.fori_loop(0, NCHK, chunk_body, 0)

        obase = pl.multiple_of(wid * BPW, BPW)
        pltpu.sync_copy(oi_v, oi_hbm.at[pl.ds(obase, BPW)])
        pltpu.sync_copy(oj_v, oj_hbm.at[pl.ds(obase, BPW)])

    return bprmf(u2, i2, j2, eu_b, ei_b)
